# Initial kernel scaffold; baseline (speedup 1.0000x reference)
import jax
import jax.numpy as jnp
from jax.experimental import pallas as pl


def kernel(center_ids, context_ids, negative_ids, W_in, W_out):
    raise NotImplementedError("write your pallas kernel here")


# speedup vs baseline: 4.8184x; 4.8184x over previous
import functools

import jax
import jax.numpy as jnp
from jax import lax
from jax.experimental import pallas as pl
from jax.experimental.pallas import tpu as pltpu
from jax.experimental.pallas import tpu_sc as plsc

VOCAB = 1000000
DIM = 64
BATCH = 16384
NNEG = 20
NPAIR = NNEG + 1

_INFO = plsc.get_sparse_core_info()
NC = _INFO.num_cores
NS = _INFO.num_subcores
NW = NC * NS
B_PER_W = BATCH // NW
C = 32
NCHUNK = B_PER_W // C
NEG_PER_CHUNK = C * NNEG
NEG_IDX_ROWS = NEG_PER_CHUNK // 128

_mesh = plsc.VectorSubcoreMesh(core_axis_name="c", subcore_axis_name="s")


@functools.partial(
    pl.kernel,
    out_type=jax.ShapeDtypeStruct((BATCH * NPAIR,), jnp.float32),
    mesh=_mesh,
    compiler_params=pltpu.CompilerParams(needs_layout_passes=False,
                                         use_tc_tiling_on_sc=False),
    scratch_types=[
        pltpu.VMEM((B_PER_W,), jnp.int32),
        pltpu.VMEM((B_PER_W,), jnp.int32),
        pltpu.VMEM((B_PER_W * NNEG,), jnp.int32),
        pltpu.VMEM((C, DIM), jnp.float32),
        pltpu.VMEM((C, DIM), jnp.float32),
        pltpu.VMEM((NEG_PER_CHUNK, DIM), jnp.float32),
        pltpu.VMEM((C * NPAIR,), jnp.float32),
        pltpu.SemaphoreType.DMA,
    ],
)
def _sc_dots(cen_hbm, ctx_hbm, neg_hbm, win_hbm, wout_hbm, out_hbm,
             cen_i, ctx_i, neg_i, cen_v, ctx_v, neg_v, out_v, sem):
    wid = lax.axis_index("s") * NC + lax.axis_index("c")

    pltpu.sync_copy(cen_hbm.at[pl.ds(wid * B_PER_W, B_PER_W)], cen_i)
    pltpu.sync_copy(ctx_hbm.at[pl.ds(wid * B_PER_W, B_PER_W)], ctx_i)
    pltpu.sync_copy(neg_hbm.at[pl.ds(wid * B_PER_W * NNEG, B_PER_W * NNEG)],
                    neg_i)

    def chunk_body(t, _):
        base = wid * B_PER_W + t * C

        cps = [
            pltpu.async_copy(win_hbm.at[cen_i.at[pl.ds(t * C, C)]],
                             cen_v, sem),
            pltpu.async_copy(wout_hbm.at[ctx_i.at[pl.ds(t * C, C)]],
                             ctx_v, sem),
        ]
        for q in range(NEG_IDX_ROWS):
            cps.append(pltpu.async_copy(
                wout_hbm.at[neg_i.at[pl.ds(t * NEG_PER_CHUNK + q * 128, 128)]],
                neg_v.at[pl.ds(q * 128, 128)], sem))
        for cp in cps:
            cp.wait()

        lane = lax.broadcasted_iota(jnp.int32, (16,), 0)
        last_lane = lane == 15

        def elem_body(i, _):
            c = [cen_v[i, pl.ds(k * 16, 16)] for k in range(DIM // 16)]

            def emit_dot(other_ref, row, slot):
                y = [other_ref[row, pl.ds(k * 16, 16)]
                     for k in range(DIM // 16)]
                p = (c[0] * y[0] + c[1] * y[1]) + (c[2] * y[2] + c[3] * y[3])
                s = plsc.cumsum(p)
                plsc.store_scatter(out_v, [jnp.full((16,), slot, jnp.int32)],
                                   s, mask=last_lane)

            emit_dot(ctx_v, i, i * NPAIR)
            for n in range(NNEG):
                emit_dot(neg_v, i * NNEG + n, i * NPAIR + (n + 1))
            return 0

        lax.fori_loop(0, C, elem_body, 0)
        pltpu.sync_copy(out_v, out_hbm.at[pl.ds(base * NPAIR, C * NPAIR)])
        return 0

    lax.fori_loop(0, NCHUNK, chunk_body, 0)


def _tc_loss_body(dots_ref, out_ref):
    x = dots_ref[...]
    rows, cols = x.shape
    flat = (lax.broadcasted_iota(jnp.int32, (rows, cols), 0) * cols
            + lax.broadcasted_iota(jnp.int32, (rows, cols), 1))
    v = jnp.where(flat % NPAIR == 0, x, -x)
    ls = -(jnp.maximum(-v, 0.0) + jnp.log1p(jnp.exp(-jnp.abs(v))))
    out_ref[...] = jnp.reshape(-jnp.sum(ls) / BATCH, (1, 1))


def kernel(center_ids, context_ids, negative_ids, W_in, W_out):
    neg_flat = negative_ids.reshape(BATCH * NNEG)
    dots = _sc_dots(center_ids, context_ids, neg_flat, W_in, W_out)
    dots2d = dots.reshape(BATCH * NPAIR // 128, 128)
    loss = pl.pallas_call(
        _tc_loss_body,
        out_shape=jax.ShapeDtypeStruct((1, 1), jnp.float32),
    )(dots2d)
    return loss[0, 0]
